# baseline (speedup 1.0000x reference)
import functools

import jax
import jax.numpy as jnp
from jax import lax
from jax.experimental import pallas as pl
from jax.experimental.pallas import tpu as pltpu
from jax.experimental.pallas import tpu_sc as plsc

NC = 2
NS = 16
NW = NC * NS
B = 4
NF = 4096
NCRS = 1024
NG = 4096
SLOTS = NW // B

L = 16
QB = 4
U = 4


def _norms(xr, yr, zr, nr, n):
    def body(i, carry):
        off = i * L
        x = xr[pl.ds(off, L)]
        y = yr[pl.ds(off, L)]
        z = zr[pl.ds(off, L)]
        nr[pl.ds(off, L)] = x * x + y * y + z * z
        return carry
    lax.fori_loop(0, n // L, body, 0)


def _dir_sum(qx, qy, qz, qn, q0, nq, tx, ty, tz, tn, nt):
    def qgroup(qg, acc):
        qbase = q0 + qg * L
        gqx = qx[pl.ds(qbase, L)]
        gqy = qy[pl.ds(qbase, L)]
        gqz = qz[pl.ds(qbase, L)]
        gqn = qn[pl.ds(qbase, L)]

        for sb in range(L // QB):
            qm2 = []
            for q in range(QB):
                i = sb * QB + q
                qm2.append((jnp.full((L,), gqx[i]) * -2.0,
                            jnp.full((L,), gqy[i]) * -2.0,
                            jnp.full((L,), gqz[i]) * -2.0))

            def tchunk(tc, carry):
                ms = list(carry)
                base = tc * (U * L)
                for u in range(U):
                    off = base + u * L
                    vx = tx[pl.ds(off, L)]
                    vy = ty[pl.ds(off, L)]
                    vz = tz[pl.ds(off, L)]
                    vn = tn[pl.ds(off, L)]
                    for q in range(QB):
                        ax, ay, az = qm2[q]
                        d = vn + vx * ax + vy * ay + vz * az
                        ms[q] = jnp.minimum(ms[q], d)
                return tuple(ms)

            init = tuple(jnp.full((L,), 1e30, jnp.float32)
                         for _ in range(QB))
            ms = lax.fori_loop(0, nt // (U * L), tchunk, init)
            for q in range(QB):
                acc = acc + jnp.min(ms[q]) + gqn[sb * QB + q]
        return acc

    return lax.fori_loop(0, nq // L, qgroup, jnp.float32(0.0))


def _sc_body(gt_hbm, fine_hbm, crs_hbm, out_hbm,
             gx, gy, gz, fx, fy, fz, cx, cy, cz,
             gn, fn, cn, ob, sem):
    wid = lax.axis_index("s") * NC + lax.axis_index("c")
    b = wid // SLOTS
    s = wid % SLOTS

    copies = [
        pltpu.async_copy(gt_hbm.at[b * 3 + 0], gx, sem),
        pltpu.async_copy(gt_hbm.at[b * 3 + 1], gy, sem),
        pltpu.async_copy(gt_hbm.at[b * 3 + 2], gz, sem),
        pltpu.async_copy(fine_hbm.at[b * 3 + 0], fx, sem),
        pltpu.async_copy(fine_hbm.at[b * 3 + 1], fy, sem),
        pltpu.async_copy(fine_hbm.at[b * 3 + 2], fz, sem),
        pltpu.async_copy(crs_hbm.at[b * 3 + 0], cx, sem),
        pltpu.async_copy(crs_hbm.at[b * 3 + 1], cy, sem),
        pltpu.async_copy(crs_hbm.at[b * 3 + 2], cz, sem),
    ]
    for c in copies:
        c.wait()

    _norms(gx, gy, gz, gn, NG)
    _norms(fx, fy, fz, fn, NF)
    _norms(cx, cy, cz, cn, NCRS)

    nf_s = NF // SLOTS
    ng_s = NG // SLOTS
    nc_s = NCRS // SLOTS

    s_fg = _dir_sum(fx, fy, fz, fn, s * nf_s, nf_s, gx, gy, gz, gn, NG)
    s_gf = _dir_sum(gx, gy, gz, gn, s * ng_s, ng_s, fx, fy, fz, fn, NF)
    s_cg = _dir_sum(cx, cy, cz, cn, s * nc_s, nc_s, gx, gy, gz, gn, NG)
    s_gc = _dir_sum(gx, gy, gz, gn, s * ng_s, ng_s, cx, cy, cz, cn, NCRS)

    lane = lax.iota(jnp.int32, L)
    v = jnp.where(lane == 0, s_fg,
                  jnp.where(lane == 1, s_gf,
                            jnp.where(lane == 2, s_cg,
                                      jnp.where(lane == 3, s_gc, 0.0))))
    ob[...] = v
    pltpu.sync_copy(ob, out_hbm.at[wid])


@jax.jit
def kernel(coarse, fine, gt, alpha):
    gt2 = gt.reshape(B * 3, NG)
    fine2 = jnp.transpose(fine, (0, 2, 1)).reshape(B * 3, NF)
    crs2 = jnp.transpose(coarse, (0, 2, 1)).reshape(B * 3, NCRS)

    mesh = plsc.VectorSubcoreMesh(core_axis_name="c", subcore_axis_name="s")
    run = functools.partial(
        pl.kernel,
        mesh=mesh,
        compiler_params=pltpu.CompilerParams(needs_layout_passes=False),
        out_type=jax.ShapeDtypeStruct((NW, L), jnp.float32),
        scratch_types=[
            pltpu.VMEM((NG,), jnp.float32),
            pltpu.VMEM((NG,), jnp.float32),
            pltpu.VMEM((NG,), jnp.float32),
            pltpu.VMEM((NF,), jnp.float32),
            pltpu.VMEM((NF,), jnp.float32),
            pltpu.VMEM((NF,), jnp.float32),
            pltpu.VMEM((NCRS,), jnp.float32),
            pltpu.VMEM((NCRS,), jnp.float32),
            pltpu.VMEM((NCRS,), jnp.float32),
            pltpu.VMEM((NG,), jnp.float32),
            pltpu.VMEM((NF,), jnp.float32),
            pltpu.VMEM((NCRS,), jnp.float32),
            pltpu.VMEM((L,), jnp.float32),
            pltpu.SemaphoreType.DMA,
        ],
    )(_sc_body)
    partial = run(gt2, fine2, crs2)

    p = partial.reshape(B, SLOTS, L)[:, :, :4].sum(axis=1)
    cham_fine = p[:, 0] / NF + p[:, 1] / NG
    cham_coarse = p[:, 2] / NCRS + p[:, 3] / NG
    loss_fine = jnp.mean(cham_fine)
    loss_coarse = jnp.mean(cham_coarse)
    loss = loss_coarse + alpha * loss_fine
    return (loss, loss_coarse, loss_fine)


# speedup vs baseline: 1.2308x; 1.2308x over previous
import functools

import jax
import jax.numpy as jnp
from jax import lax
from jax.experimental import pallas as pl
from jax.experimental.pallas import tpu as pltpu
from jax.experimental.pallas import tpu_sc as plsc

NC = 2
NS = 16
NW = NC * NS
B = 4
NF = 4096
NCRS = 1024
NG = 4096
SLOTS = NW // B

L = 16
QB = 4
U = 4
BIG = 1e30


def _norms(xr, yr, zr, nr, n):
    def body(i, carry):
        off = i * L
        x = xr[pl.ds(off, L)]
        y = yr[pl.ds(off, L)]
        z = zr[pl.ds(off, L)]
        nr[pl.ds(off, L)] = x * x + y * y + z * z
        return carry
    lax.fori_loop(0, n // L, body, 0)


def _fill(ref, n, val):
    v = jnp.full((L,), val, jnp.float32)
    def body(i, carry):
        ref[pl.ds(i * L, L)] = v
        return carry
    lax.fori_loop(0, n // L, body, 0)


def _fused_pass(qx, qy, qz, qn, q0, nq, tx, ty, tz, tn, nt, cm):
    def qgroup(qg, acc):
        qbase = q0 + qg * L
        gqx = qx[pl.ds(qbase, L)]
        gqy = qy[pl.ds(qbase, L)]
        gqz = qz[pl.ds(qbase, L)]
        gqn = qn[pl.ds(qbase, L)]

        for sb in range(L // QB):
            qm2 = []
            qnb = []
            for q in range(QB):
                i = sb * QB + q
                qm2.append((jnp.full((L,), gqx[i]) * -2.0,
                            jnp.full((L,), gqy[i]) * -2.0,
                            jnp.full((L,), gqz[i]) * -2.0))
                qnb.append(jnp.full((L,), gqn[i]))

            def tchunk(tc, carry):
                ms = list(carry)
                base = tc * (U * L)
                for u in range(U):
                    off = base + u * L
                    vx = tx[pl.ds(off, L)]
                    vy = ty[pl.ds(off, L)]
                    vz = tz[pl.ds(off, L)]
                    vn = tn[pl.ds(off, L)]
                    ds = []
                    for q in range(QB):
                        ax, ay, az = qm2[q]
                        d = vn + vx * ax + vy * ay + vz * az
                        ms[q] = jnp.minimum(ms[q], d)
                        ds.append(d + qnb[q])
                    e = jnp.minimum(jnp.minimum(ds[0], ds[1]),
                                    jnp.minimum(ds[2], ds[3]))
                    cm[pl.ds(off, L)] = jnp.minimum(cm[pl.ds(off, L)], e)
                return tuple(ms)

            init = tuple(jnp.full((L,), BIG, jnp.float32)
                         for _ in range(QB))
            ms = lax.fori_loop(0, nt // (U * L), tchunk, init)
            for q in range(QB):
                acc = acc + jnp.min(ms[q]) + gqn[sb * QB + q]
        return acc

    return lax.fori_loop(0, nq // L, qgroup, jnp.float32(0.0))


def _merge_cols(shared, row0, slot, mb, n_per_slot):
    col0 = slot * n_per_slot
    for r in range(SLOTS):
        pltpu.sync_copy(shared.at[row0 + r, pl.ds(col0, n_per_slot)],
                        mb.at[r])

    def chunk(c, acc):
        off = c * L
        m = mb[0, pl.ds(off, L)]
        for r in range(1, SLOTS):
            m = jnp.minimum(m, mb[r, pl.ds(off, L)])
        return acc + m

    sv = lax.fori_loop(0, n_per_slot // L, chunk,
                       jnp.zeros((L,), jnp.float32))
    return jnp.sum(sv)


def _sc_body(gt_hbm, fine_hbm, crs_hbm, out_hbm,
             gx, gy, gz, fx, fy, fz, cx, cy, cz,
             gn, fn, cn, cmf, cmc, mbf, mbc, ob,
             shf, shc, sem):
    core = lax.axis_index("c")
    sub = lax.axis_index("s")
    wid = core * NS + sub
    b = wid // SLOTS
    slot = wid % SLOTS
    row0 = (sub // SLOTS) * SLOTS

    copies = [
        pltpu.async_copy(gt_hbm.at[b * 3 + 0], gx, sem),
        pltpu.async_copy(gt_hbm.at[b * 3 + 1], gy, sem),
        pltpu.async_copy(gt_hbm.at[b * 3 + 2], gz, sem),
        pltpu.async_copy(fine_hbm.at[b * 3 + 0], fx, sem),
        pltpu.async_copy(fine_hbm.at[b * 3 + 1], fy, sem),
        pltpu.async_copy(fine_hbm.at[b * 3 + 2], fz, sem),
        pltpu.async_copy(crs_hbm.at[b * 3 + 0], cx, sem),
        pltpu.async_copy(crs_hbm.at[b * 3 + 1], cy, sem),
        pltpu.async_copy(crs_hbm.at[b * 3 + 2], cz, sem),
    ]
    for c in copies:
        c.wait()

    _norms(gx, gy, gz, gn, NG)
    _norms(fx, fy, fz, fn, NF)
    _norms(cx, cy, cz, cn, NCRS)
    _fill(cmf, NG, BIG)
    _fill(cmc, NG, BIG)

    nf_s = NF // SLOTS
    nc_s = NCRS // SLOTS

    s_fg = _fused_pass(fx, fy, fz, fn, slot * nf_s, nf_s,
                       gx, gy, gz, gn, NG, cmf)
    s_cg = _fused_pass(cx, cy, cz, cn, slot * nc_s, nc_s,
                       gx, gy, gz, gn, NG, cmc)

    pltpu.sync_copy(cmf, shf.at[sub])
    pltpu.sync_copy(cmc, shc.at[sub])
    plsc.subcore_barrier()

    ng_s = NG // SLOTS
    s_gf = _merge_cols(shf, row0, slot, mbf, ng_s)
    s_gc = _merge_cols(shc, row0, slot, mbc, ng_s)

    lane = lax.iota(jnp.int32, L)
    v = jnp.where(lane == 0, s_fg,
                  jnp.where(lane == 1, s_gf,
                            jnp.where(lane == 2, s_cg,
                                      jnp.where(lane == 3, s_gc, 0.0))))
    ob[...] = v
    pltpu.sync_copy(ob, out_hbm.at[wid])


@jax.jit
def kernel(coarse, fine, gt, alpha):
    gt2 = gt.reshape(B * 3, NG)
    fine2 = jnp.transpose(fine, (0, 2, 1)).reshape(B * 3, NF)
    crs2 = jnp.transpose(coarse, (0, 2, 1)).reshape(B * 3, NCRS)

    mesh = plsc.VectorSubcoreMesh(core_axis_name="c", subcore_axis_name="s")
    run = functools.partial(
        pl.kernel,
        mesh=mesh,
        compiler_params=pltpu.CompilerParams(needs_layout_passes=False),
        out_type=jax.ShapeDtypeStruct((NW, L), jnp.float32),
        scratch_types=[
            pltpu.VMEM((NG,), jnp.float32),
            pltpu.VMEM((NG,), jnp.float32),
            pltpu.VMEM((NG,), jnp.float32),
            pltpu.VMEM((NF,), jnp.float32),
            pltpu.VMEM((NF,), jnp.float32),
            pltpu.VMEM((NF,), jnp.float32),
            pltpu.VMEM((NCRS,), jnp.float32),
            pltpu.VMEM((NCRS,), jnp.float32),
            pltpu.VMEM((NCRS,), jnp.float32),
            pltpu.VMEM((NG,), jnp.float32),
            pltpu.VMEM((NF,), jnp.float32),
            pltpu.VMEM((NCRS,), jnp.float32),
            pltpu.VMEM((NG,), jnp.float32),
            pltpu.VMEM((NG,), jnp.float32),
            pltpu.VMEM((SLOTS, NG // SLOTS), jnp.float32),
            pltpu.VMEM((SLOTS, NG // SLOTS), jnp.float32),
            pltpu.VMEM((L,), jnp.float32),
            pltpu.VMEM_SHARED((NS, NG), jnp.float32),
            pltpu.VMEM_SHARED((NS, NG), jnp.float32),
            pltpu.SemaphoreType.DMA,
        ],
    )(_sc_body)
    partial = run(gt2, fine2, crs2)

    p = partial.reshape(B, SLOTS, L)[:, :, :4].sum(axis=1)
    cham_fine = p[:, 0] / NF + p[:, 1] / NG
    cham_coarse = p[:, 2] / NCRS + p[:, 3] / NG
    loss_fine = jnp.mean(cham_fine)
    loss_coarse = jnp.mean(cham_coarse)
    loss = loss_coarse + alpha * loss_fine
    return (loss, loss_coarse, loss_fine)
